# baseline (speedup 1.0000x reference)
import jax
import jax.numpy as jnp
import numpy as np
from jax.experimental import pallas as pl
from jax.experimental.pallas import tpu as pltpu

_NL = 8
_LEAF = 128
_HID = _NL * _LEAF
_LATENT = 384
_SEQ = 196
_IMB = 4


def _tree3_consts(C, E):
    Cn, En = np.asarray(C), np.asarray(E)
    T3 = np.zeros((75, 3 * _NL), np.float32)
    E3 = np.zeros((3 * _NL, 3 * _HID), np.float32)
    for t in range(3):
        r, c = 8 * t, 8 * t
        for d in range(3):
            T3[24 * d + r:24 * d + r + 8, c:c + 8] = Cn[8 * d:8 * d + 8]
            T3[72 + d, c:c + 8] = Cn[24 + d]
        E3[r:r + 8, t * _HID:(t + 1) * _HID] = En
    return jnp.asarray(T3), jnp.asarray(E3)


def _tree_consts():
    C = np.zeros((32, _NL), np.float32)
    E = np.zeros((_NL, _HID), np.float32)
    for l in range(_NL):
        b2, b1, b0 = (l >> 2) & 1, (l >> 1) & 1, l & 1
        C[0 + 0, l] = 1.0 if b2 else -1.0
        C[8 + 1 + b2, l] = 1.0 if b1 else -1.0
        C[16 + 3 + (l >> 1), l] = 1.0 if b0 else -1.0
        C[24, l] = 0.0 if b2 else 1.0
        C[25, l] = 0.0 if b1 else 1.0
        C[26, l] = 0.0 if b0 else 1.0
        E[l, l * _LEAF:(l + 1) * _LEAF] = 1.0
    return C, E


def _mixmul(h, mix):
    h16 = h.astype(jnp.bfloat16)
    m16 = mix.astype(jnp.bfloat16)
    rows = h.shape[0]
    parts = [h16[:, i * _LEAF:(i + 1) * _LEAF] *
             jnp.broadcast_to(m16[:, i:i + 1], (rows, _LEAF))
             for i in range(mix.shape[1])]
    return jnp.concatenate(parts, axis=1)


def _dot16(a, b):
    return jnp.dot(a.astype(jnp.bfloat16), b, preferred_element_type=jnp.float32)


def _fff(x, tree, E, nwT, nb, W1, b1, W2, b2, act):
    e = jax.nn.sigmoid(_dot16(x, nwT) + nb)
    t0 = jnp.dot(e, tree[0:8]) + tree[24:25]
    t1 = jnp.dot(e, tree[8:16]) + tree[25:26]
    t2 = jnp.dot(e, tree[16:24]) + tree[26:27]
    mix = t0 * t1 * t2
    h = act(_dot16(x, W1) + b1)
    hm = _mixmul(h, mix)
    return jnp.dot(hm, W2, preferred_element_type=jnp.float32) + jnp.dot(mix, b2)


def _gelu(z):
    return 0.5 * z * (1.0 + jax.lax.erf(z * (2.0 ** -0.5)))


def _layernorm(x):
    mean = jnp.mean(x, axis=-1, keepdims=True)
    d = x - mean
    var = jnp.sum(d * d, axis=-1, keepdims=True) / (_LATENT - 1)
    std = jnp.sqrt(var)
    return d / jnp.sqrt(std + 1e-05)


def _mega_body(img_ref, pe_ref, tree_ref, E_ref, T3_ref, E3_ref,
               tnwT_ref, tnb_ref, tW1_ref, tb1_ref, tW2_ref, tb2_ref,
               inwT_ref, inb_ref, iW1_ref, ib1_ref, iW2_ref, ib2_ref,
               qnwT_ref, qnb_ref, qW1_ref, qb1_ref, qb2d_ref,
               out_ref):
    tree = tree_ref[...]
    E = E_ref[...]
    T3 = T3_ref[...]
    E3 = E3_ref[...]

    def fff_j(z, j, act):
        return _fff(z, tree, E, inwT_ref[j], inb_ref[j], iW1_ref[j],
                    ib1_ref[j], iW2_ref[j], ib2_ref[j], act)

    X = img_ref[...].astype(jnp.bfloat16)
    P = [X[i].reshape(3, 14, 16, 14, 16).transpose(0, 1, 3, 2, 4)
         .reshape(3, _SEQ, 256) for i in range(_IMB)]
    P = jnp.concatenate(P, axis=1)
    ea = jnp.dot(P[0], tnwT_ref[0], preferred_element_type=jnp.float32)
    ha = jnp.dot(P[0], tW1_ref[0], preferred_element_type=jnp.float32)
    for c in (1, 2):
        ea = ea + jnp.dot(P[c], tnwT_ref[c], preferred_element_type=jnp.float32)
        ha = ha + jnp.dot(P[c], tW1_ref[c], preferred_element_type=jnp.float32)
    e = jax.nn.sigmoid(ea + tnb_ref[...])
    t0 = jnp.dot(e, tree[0:8]) + tree[24:25]
    t1 = jnp.dot(e, tree[8:16]) + tree[25:26]
    t2 = jnp.dot(e, tree[16:24]) + tree[26:27]
    mix = t0 * t1 * t2
    h = jax.nn.relu(ha + tb1_ref[...])
    hm = _mixmul(h, mix)
    x = jnp.dot(hm, tW2_ref[...], preferred_element_type=jnp.float32)
    x = x + jnp.dot(mix, tb2_ref[...])
    x = x + pe_ref[...]
    for bb, base in enumerate((0, 4)):
        xn = _layernorm(x)
        e3 = jax.nn.sigmoid(_dot16(xn, qnwT_ref[bb]) + qnb_ref[bb])
        t0 = jnp.dot(e3, T3[0:24]) + T3[72:73]
        t1 = jnp.dot(e3, T3[24:48]) + T3[73:74]
        t2 = jnp.dot(e3, T3[48:72]) + T3[74:75]
        mix3 = t0 * t1 * t2
        H = jax.nn.relu(_dot16(xn, qW1_ref[bb]) + qb1_ref[bb])
        HM = _mixmul(H, mix3)
        z = jnp.dot(mix3, qb2d_ref[bb])
        q = (jnp.dot(HM[:, 0 * _HID:1 * _HID], iW2_ref[base + 0],
                     preferred_element_type=jnp.float32)
             + z[:, 0 * _LATENT:1 * _LATENT]).astype(jnp.bfloat16)
        k = (jnp.dot(HM[:, 1 * _HID:2 * _HID], iW2_ref[base + 1],
                     preferred_element_type=jnp.float32)
             + z[:, 1 * _LATENT:2 * _LATENT]).astype(jnp.bfloat16)
        v = (jnp.dot(HM[:, 2 * _HID:3 * _HID], iW2_ref[base + 2],
                     preferred_element_type=jnp.float32)
             + z[:, 2 * _LATENT:3 * _LATENT]).astype(jnp.bfloat16)
        avs = []
        for i in range(_IMB):
            sl = slice(i * _SEQ, (i + 1) * _SEQ)
            s = jax.lax.dot_general(q[sl], k[sl], (((1,), (1,)), ((), ())),
                                    preferred_element_type=jnp.float32)
            s = s / (_LATENT ** 0.5)
            m = jnp.max(s, axis=-1, keepdims=True)
            p = jnp.exp(s - m)
            att = p / jnp.sum(p, axis=-1, keepdims=True)
            avs.append(_dot16(att, v[sl]))
        x1 = xn + jnp.concatenate(avs, axis=0)
        x = x1 + fff_j(x1, base + 3, _gelu)
    out_ref[...] = jnp.mean(x.reshape(_IMB, _SEQ, _LATENT), axis=1,
                            keepdims=True)


def _head_body(x_ref, tree_ref, E_ref, nwT_ref, nb_ref, W1_ref, b1_ref,
               W2_ref, b2_ref, out_ref):
    out_ref[...] = _fff(x_ref[...], tree_ref[...], E_ref[...], nwT_ref[...],
                        nb_ref[...], W1_ref[...], b1_ref[...], W2_ref[...],
                        b2_ref[...], jax.nn.relu)


def _full(arr):
    nd = len(arr.shape)
    return pl.BlockSpec(arr.shape, lambda i, _nd=nd: (0,) * _nd)


def _prep_fff(nw, nb, w1, b1, w2, b2, in_w):
    nwT = jnp.swapaxes(jnp.pad(nw, [(0, 0)] * (nw.ndim - 2) + [(0, 1), (0, 0)]),
                       -1, -2)
    nbr = jnp.swapaxes(jnp.pad(nb, [(0, 0)] * (nb.ndim - 2) + [(0, 1), (0, 0)]),
                       -1, -2)
    W1 = jnp.swapaxes(w1, -3, -2).reshape(w1.shape[:-3] + (in_w, _HID))
    b1r = b1.reshape(b1.shape[:-2] + (1, _HID))
    W2 = w2.reshape(w2.shape[:-3] + (_HID, w2.shape[-1]))
    return (nwT.astype(jnp.bfloat16), nbr, W1.astype(jnp.bfloat16), b1r,
            W2.astype(jnp.bfloat16), b2)


def kernel(imgs, tok_nw, tok_nb, tok_w1, tok_b1, tok_w2, tok_b2,
           inn_nw, inn_nb, inn_w1, inn_b1, inn_w2, inn_b2,
           out_nw, out_nb, out_w1, out_b1, out_w2, out_b2):
    B, C, H, W = imgs.shape
    in_w = C * 16 * 16

    treeN, EN = _tree_consts()
    T3N, E3N = _tree3_consts(treeN, EN)
    tree, E = jnp.asarray(treeN), jnp.asarray(EN)
    T3, E3 = jnp.asarray(T3N), jnp.asarray(E3N)

    s = jnp.arange(_SEQ, dtype=jnp.float32)
    pe = jnp.where((jnp.arange(_LATENT) % 2 == 0)[None, :],
                   jnp.sin(s)[:, None], jnp.cos(s)[:, None])

    tok = list(_prep_fff(tok_nw, tok_nb, tok_w1, tok_b1, tok_w2, tok_b2, in_w))
    tok[0] = tok[0].reshape(C, in_w // C, _NL)
    tok[2] = tok[2].reshape(C, in_w // C, _HID)
    inn = _prep_fff(inn_nw, inn_nb, inn_w1, inn_b1, inn_w2, inn_b2, _LATENT)
    out = _prep_fff(out_nw, out_nb, out_w1, out_b1, out_w2, out_b2, _LATENT)

    qnwT = jnp.stack([jnp.concatenate([inn[0][j] for j in ids], axis=-1)
                      for ids in ((0, 1, 2), (4, 5, 6))])
    qnb = jnp.stack([jnp.concatenate([inn[1][j] for j in ids], axis=-1)
                     for ids in ((0, 1, 2), (4, 5, 6))])
    qW1 = jnp.stack([jnp.concatenate([inn[2][j] for j in ids], axis=-1)
                     for ids in ((0, 1, 2), (4, 5, 6))])
    qb1 = jnp.stack([jnp.concatenate([inn[3][j] for j in ids], axis=-1)
                     for ids in ((0, 1, 2), (4, 5, 6))])
    z24 = jnp.zeros((_NL, _LATENT), jnp.float32)
    qb2d = jnp.stack(
        [jnp.block([[inn[5][i0], z24, z24],
                    [z24, inn[5][i0 + 1], z24],
                    [z24, z24, inn[5][i0 + 2]]]) for i0 in (0, 4)])

    pe4 = jnp.tile(pe, (_IMB, 1))
    xm = pl.pallas_call(
        _mega_body,
        grid=(B // _IMB,),
        in_specs=[pl.BlockSpec((_IMB, C, H, W), lambda i: (i, 0, 0, 0)),
                  _full(pe4), _full(tree), _full(E), _full(T3), _full(E3)] +
                 [_full(a) for a in tok] + [_full(a) for a in inn] +
                 [_full(a) for a in (qnwT, qnb, qW1, qb1, qb2d)],
        out_specs=pl.BlockSpec((_IMB, 1, _LATENT), lambda i: (i, 0, 0)),
        out_shape=jax.ShapeDtypeStruct((B, 1, _LATENT), jnp.float32),
        compiler_params=pltpu.CompilerParams(
            dimension_semantics=("parallel",)),
    )(imgs, pe4, tree, E, T3, E3, *tok, *inn, qnwT, qnb, qW1, qb1, qb2d)
    xm = xm.reshape(B, _LATENT)

    y = pl.pallas_call(
        _head_body,
        grid=(1,),
        in_specs=[_full(xm), _full(tree), _full(E)] + [_full(a) for a in out],
        out_specs=_full(jax.ShapeDtypeStruct((B, out_w2.shape[-1]), jnp.float32)),
        out_shape=jax.ShapeDtypeStruct((B, out_w2.shape[-1]), jnp.float32),
    )(xm, tree, E, *out)
    return y


# speedup vs baseline: 1.0225x; 1.0225x over previous
import jax
import jax.numpy as jnp
import numpy as np
from jax.experimental import pallas as pl
from jax.experimental.pallas import tpu as pltpu

_NL = 8
_LEAF = 128
_HID = _NL * _LEAF
_LATENT = 384
_SEQ = 196
_IMB = 2


def _tree3_consts(C, E):
    Cn, En = np.asarray(C), np.asarray(E)
    T3 = np.zeros((75, 3 * _NL), np.float32)
    E3 = np.zeros((3 * _NL, 3 * _HID), np.float32)
    for t in range(3):
        r, c = 8 * t, 8 * t
        for d in range(3):
            T3[24 * d + r:24 * d + r + 8, c:c + 8] = Cn[8 * d:8 * d + 8]
            T3[72 + d, c:c + 8] = Cn[24 + d]
        E3[r:r + 8, t * _HID:(t + 1) * _HID] = En
    return jnp.asarray(T3), jnp.asarray(E3)


def _tree_consts():
    C = np.zeros((32, _NL), np.float32)
    E = np.zeros((_NL, _HID), np.float32)
    for l in range(_NL):
        b2, b1, b0 = (l >> 2) & 1, (l >> 1) & 1, l & 1
        C[0 + 0, l] = 1.0 if b2 else -1.0
        C[8 + 1 + b2, l] = 1.0 if b1 else -1.0
        C[16 + 3 + (l >> 1), l] = 1.0 if b0 else -1.0
        C[24, l] = 0.0 if b2 else 1.0
        C[25, l] = 0.0 if b1 else 1.0
        C[26, l] = 0.0 if b0 else 1.0
        E[l, l * _LEAF:(l + 1) * _LEAF] = 1.0
    return C, E


def _mixmul(h, mix):
    h16 = h.astype(jnp.bfloat16)
    m16 = mix.astype(jnp.bfloat16)
    rows = h.shape[0]
    parts = [h16[:, i * _LEAF:(i + 1) * _LEAF] *
             jnp.broadcast_to(m16[:, i:i + 1], (rows, _LEAF))
             for i in range(mix.shape[1])]
    return jnp.concatenate(parts, axis=1)


def _dot16(a, b):
    return jnp.dot(a.astype(jnp.bfloat16), b, preferred_element_type=jnp.float32)


def _fff(x, tree, E, nwT, nb, W1, b1, W2, b2, act):
    e = jax.nn.sigmoid(_dot16(x, nwT) + nb)
    t0 = jnp.dot(e, tree[0:8]) + tree[24:25]
    t1 = jnp.dot(e, tree[8:16]) + tree[25:26]
    t2 = jnp.dot(e, tree[16:24]) + tree[26:27]
    mix = t0 * t1 * t2
    h = act(_dot16(x, W1) + b1)
    hm = _mixmul(h, mix)
    return jnp.dot(hm, W2, preferred_element_type=jnp.float32) + jnp.dot(mix, b2)


def _gelu(z):
    return 0.5 * z * (1.0 + jax.lax.erf(z * (2.0 ** -0.5)))


def _layernorm(x):
    mean = jnp.mean(x, axis=-1, keepdims=True)
    d = x - mean
    var = jnp.sum(d * d, axis=-1, keepdims=True) / (_LATENT - 1)
    std = jnp.sqrt(var)
    return d / jnp.sqrt(std + 1e-05)


def _mega_body(img_ref, pe_ref, tree_ref, E_ref, T3_ref, E3_ref,
               tnwT_ref, tnb_ref, tW1_ref, tb1_ref, tW2_ref, tb2_ref,
               inwT_ref, inb_ref, iW1_ref, ib1_ref, iW2_ref, ib2_ref,
               qnwT_ref, qnb_ref, qW1_ref, qb1_ref, qb2d_ref,
               out_ref):
    tree = tree_ref[...]
    E = E_ref[...]
    T3 = T3_ref[...]
    E3 = E3_ref[...]

    def fff_j(z, j, act):
        return _fff(z, tree, E, inwT_ref[j], inb_ref[j], iW1_ref[j],
                    ib1_ref[j], iW2_ref[j], ib2_ref[j], act)

    X = img_ref[...].astype(jnp.bfloat16)
    P = [X[i].reshape(3, 14, 16, 14, 16).transpose(0, 3, 1, 2, 4)
         .reshape(3, _SEQ, 256) for i in range(_IMB)]
    P = jnp.concatenate(P, axis=1)
    ea = jnp.dot(P[0], tnwT_ref[0], preferred_element_type=jnp.float32)
    ha = jnp.dot(P[0], tW1_ref[0], preferred_element_type=jnp.float32)
    for c in (1, 2):
        ea = ea + jnp.dot(P[c], tnwT_ref[c], preferred_element_type=jnp.float32)
        ha = ha + jnp.dot(P[c], tW1_ref[c], preferred_element_type=jnp.float32)
    e = jax.nn.sigmoid(ea + tnb_ref[...])
    t0 = jnp.dot(e, tree[0:8]) + tree[24:25]
    t1 = jnp.dot(e, tree[8:16]) + tree[25:26]
    t2 = jnp.dot(e, tree[16:24]) + tree[26:27]
    mix = t0 * t1 * t2
    h = jax.nn.relu(ha + tb1_ref[...])
    hm = _mixmul(h, mix)
    x = jnp.dot(hm, tW2_ref[...], preferred_element_type=jnp.float32)
    x = x + jnp.dot(mix, tb2_ref[...])
    x = x + pe_ref[...]
    for bb, base in enumerate((0, 4)):
        xn = _layernorm(x)
        e3 = jax.nn.sigmoid(_dot16(xn, qnwT_ref[bb]) + qnb_ref[bb])
        t0 = jnp.dot(e3, T3[0:24]) + T3[72:73]
        t1 = jnp.dot(e3, T3[24:48]) + T3[73:74]
        t2 = jnp.dot(e3, T3[48:72]) + T3[74:75]
        mix3 = t0 * t1 * t2
        H = jax.nn.relu(_dot16(xn, qW1_ref[bb]) + qb1_ref[bb])
        HM = _mixmul(H, mix3)
        z = jnp.dot(mix3, qb2d_ref[bb])
        q = (jnp.dot(HM[:, 0 * _HID:1 * _HID], iW2_ref[base + 0],
                     preferred_element_type=jnp.float32)
             + z[:, 0 * _LATENT:1 * _LATENT]).astype(jnp.bfloat16)
        k = (jnp.dot(HM[:, 1 * _HID:2 * _HID], iW2_ref[base + 1],
                     preferred_element_type=jnp.float32)
             + z[:, 1 * _LATENT:2 * _LATENT]).astype(jnp.bfloat16)
        v = (jnp.dot(HM[:, 2 * _HID:3 * _HID], iW2_ref[base + 2],
                     preferred_element_type=jnp.float32)
             + z[:, 2 * _LATENT:3 * _LATENT]).astype(jnp.bfloat16)
        avs = []
        for i in range(_IMB):
            sl = slice(i * _SEQ, (i + 1) * _SEQ)
            s = jax.lax.dot_general(q[sl], k[sl], (((1,), (1,)), ((), ())),
                                    preferred_element_type=jnp.float32)
            s = s / (_LATENT ** 0.5)
            m = jnp.max(s, axis=-1, keepdims=True)
            p = jnp.exp(s - m)
            att = p / jnp.sum(p, axis=-1, keepdims=True)
            avs.append(_dot16(att, v[sl]))
        x1 = xn + jnp.concatenate(avs, axis=0)
        x = x1 + fff_j(x1, base + 3, _gelu)
    out_ref[...] = jnp.mean(x.reshape(_IMB, _SEQ, _LATENT), axis=1,
                            keepdims=True)


def _head_body(x_ref, tree_ref, E_ref, nwT_ref, nb_ref, W1_ref, b1_ref,
               W2_ref, b2_ref, out_ref):
    out_ref[...] = _fff(x_ref[...], tree_ref[...], E_ref[...], nwT_ref[...],
                        nb_ref[...], W1_ref[...], b1_ref[...], W2_ref[...],
                        b2_ref[...], jax.nn.relu)


def _full(arr):
    nd = len(arr.shape)
    return pl.BlockSpec(arr.shape, lambda i, _nd=nd: (0,) * _nd)


def _prep_fff(nw, nb, w1, b1, w2, b2, in_w):
    nwT = jnp.swapaxes(jnp.pad(nw, [(0, 0)] * (nw.ndim - 2) + [(0, 1), (0, 0)]),
                       -1, -2)
    nbr = jnp.swapaxes(jnp.pad(nb, [(0, 0)] * (nb.ndim - 2) + [(0, 1), (0, 0)]),
                       -1, -2)
    W1 = jnp.swapaxes(w1, -3, -2).reshape(w1.shape[:-3] + (in_w, _HID))
    b1r = b1.reshape(b1.shape[:-2] + (1, _HID))
    W2 = w2.reshape(w2.shape[:-3] + (_HID, w2.shape[-1]))
    return (nwT.astype(jnp.bfloat16), nbr, W1.astype(jnp.bfloat16), b1r,
            W2.astype(jnp.bfloat16), b2)


def kernel(imgs, tok_nw, tok_nb, tok_w1, tok_b1, tok_w2, tok_b2,
           inn_nw, inn_nb, inn_w1, inn_b1, inn_w2, inn_b2,
           out_nw, out_nb, out_w1, out_b1, out_w2, out_b2):
    B, C, H, W = imgs.shape
    in_w = C * 16 * 16

    treeN, EN = _tree_consts()
    T3N, E3N = _tree3_consts(treeN, EN)
    tree, E = jnp.asarray(treeN), jnp.asarray(EN)
    T3, E3 = jnp.asarray(T3N), jnp.asarray(E3N)

    s = jnp.arange(_SEQ, dtype=jnp.float32)
    pe = jnp.where((jnp.arange(_LATENT) % 2 == 0)[None, :],
                   jnp.sin(s)[:, None], jnp.cos(s)[:, None])
    pe = pe[np.arange(_SEQ).reshape(14, 14).T.reshape(_SEQ)]

    tok = list(_prep_fff(tok_nw, tok_nb, tok_w1, tok_b1, tok_w2, tok_b2, in_w))
    tok[0] = tok[0].reshape(C, in_w // C, _NL)
    tok[2] = tok[2].reshape(C, in_w // C, _HID)
    inn = _prep_fff(inn_nw, inn_nb, inn_w1, inn_b1, inn_w2, inn_b2, _LATENT)
    out = _prep_fff(out_nw, out_nb, out_w1, out_b1, out_w2, out_b2, _LATENT)

    qnwT = jnp.stack([jnp.concatenate([inn[0][j] for j in ids], axis=-1)
                      for ids in ((0, 1, 2), (4, 5, 6))])
    qnb = jnp.stack([jnp.concatenate([inn[1][j] for j in ids], axis=-1)
                     for ids in ((0, 1, 2), (4, 5, 6))])
    qW1 = jnp.stack([jnp.concatenate([inn[2][j] for j in ids], axis=-1)
                     for ids in ((0, 1, 2), (4, 5, 6))])
    qb1 = jnp.stack([jnp.concatenate([inn[3][j] for j in ids], axis=-1)
                     for ids in ((0, 1, 2), (4, 5, 6))])
    z24 = jnp.zeros((_NL, _LATENT), jnp.float32)
    qb2d = jnp.stack(
        [jnp.block([[inn[5][i0], z24, z24],
                    [z24, inn[5][i0 + 1], z24],
                    [z24, z24, inn[5][i0 + 2]]]) for i0 in (0, 4)])

    pe4 = jnp.tile(pe, (_IMB, 1))
    xm = pl.pallas_call(
        _mega_body,
        grid=(B // _IMB,),
        in_specs=[pl.BlockSpec((_IMB, C, H, W), lambda i: (i, 0, 0, 0)),
                  _full(pe4), _full(tree), _full(E), _full(T3), _full(E3)] +
                 [_full(a) for a in tok] + [_full(a) for a in inn] +
                 [_full(a) for a in (qnwT, qnb, qW1, qb1, qb2d)],
        out_specs=pl.BlockSpec((_IMB, 1, _LATENT), lambda i: (i, 0, 0)),
        out_shape=jax.ShapeDtypeStruct((B, 1, _LATENT), jnp.float32),
        compiler_params=pltpu.CompilerParams(
            dimension_semantics=("parallel",)),
    )(imgs, pe4, tree, E, T3, E3, *tok, *inn, qnwT, qnb, qW1, qb1, qb2d)
    xm = xm.reshape(B, _LATENT)

    y = pl.pallas_call(
        _head_body,
        grid=(1,),
        in_specs=[_full(xm), _full(tree), _full(E)] + [_full(a) for a in out],
        out_specs=_full(jax.ShapeDtypeStruct((B, out_w2.shape[-1]), jnp.float32)),
        out_shape=jax.ShapeDtypeStruct((B, out_w2.shape[-1]), jnp.float32),
    )(xm, tree, E, *out)
    return y


# speedup vs baseline: 1.0404x; 1.0175x over previous
import jax
import jax.numpy as jnp
import numpy as np
from jax.experimental import pallas as pl
from jax.experimental.pallas import tpu as pltpu

_NL = 8
_LEAF = 128
_HID = _NL * _LEAF
_LATENT = 384
_SEQ = 196
_IMB = 2


def _tree3_consts(C, E):
    Cn, En = np.asarray(C), np.asarray(E)
    T3 = np.zeros((75, 3 * _NL), np.float32)
    E3 = np.zeros((3 * _NL, 3 * _HID), np.float32)
    for t in range(3):
        r, c = 8 * t, 8 * t
        for d in range(3):
            T3[24 * d + r:24 * d + r + 8, c:c + 8] = Cn[8 * d:8 * d + 8]
            T3[72 + d, c:c + 8] = Cn[24 + d]
        E3[r:r + 8, t * _HID:(t + 1) * _HID] = En
    return jnp.asarray(T3), jnp.asarray(E3)


def _tree_consts():
    C = np.zeros((32, _NL), np.float32)
    E = np.zeros((_NL, _HID), np.float32)
    for l in range(_NL):
        b2, b1, b0 = (l >> 2) & 1, (l >> 1) & 1, l & 1
        C[0 + 0, l] = 1.0 if b2 else -1.0
        C[8 + 1 + b2, l] = 1.0 if b1 else -1.0
        C[16 + 3 + (l >> 1), l] = 1.0 if b0 else -1.0
        C[24, l] = 0.0 if b2 else 1.0
        C[25, l] = 0.0 if b1 else 1.0
        C[26, l] = 0.0 if b0 else 1.0
        E[l, l * _LEAF:(l + 1) * _LEAF] = 1.0
    return C, E


def _mixmul(h, mix):
    h16 = h.astype(jnp.bfloat16)
    m16 = mix.astype(jnp.bfloat16)
    rows = h.shape[0]
    parts = [h16[:, i * _LEAF:(i + 1) * _LEAF] *
             jnp.broadcast_to(m16[:, i:i + 1], (rows, _LEAF))
             for i in range(mix.shape[1])]
    return jnp.concatenate(parts, axis=1)


def _dot16(a, b):
    return jnp.dot(a.astype(jnp.bfloat16), b, preferred_element_type=jnp.float32)


def _fff(x, tree, E, nwT, nb, W1, b1, W2, b2, act):
    e = jax.nn.sigmoid(_dot16(x, nwT) + nb)
    t0 = jnp.dot(e, tree[0:8]) + tree[24:25]
    t1 = jnp.dot(e, tree[8:16]) + tree[25:26]
    t2 = jnp.dot(e, tree[16:24]) + tree[26:27]
    mix = t0 * t1 * t2
    h = act(_dot16(x, W1) + b1)
    hm = _mixmul(h, mix)
    return jnp.dot(hm, W2, preferred_element_type=jnp.float32) + jnp.dot(mix, b2)


def _gelu(z):
    return 0.5 * z * (1.0 + jax.lax.erf(z * (2.0 ** -0.5)))


def _layernorm(x):
    mean = jnp.mean(x, axis=-1, keepdims=True)
    d = x - mean
    var = jnp.sum(d * d, axis=-1, keepdims=True) / (_LATENT - 1)
    std = jnp.sqrt(var)
    return d / jnp.sqrt(std + 1e-05)


def _mega_body(img_ref, pe_ref, tree_ref, E_ref, T3_ref, E3_ref,
               tnwT_ref, tnb_ref, tW1_ref, tb1_ref, tW2_ref, tb2_ref,
               inwT_ref, inb_ref, iW1_ref, ib1_ref, iW2_ref, ib2_ref,
               qnwT_ref, qnb_ref, qW1_ref, qb1_ref, qb2d_ref,
               out_ref):
    tree = tree_ref[...]
    E = E_ref[...]
    T3 = T3_ref[...]
    E3 = E3_ref[...]

    def fff_j(z, j, act):
        return _fff(z, tree, E, inwT_ref[j], inb_ref[j], iW1_ref[j],
                    ib1_ref[j], iW2_ref[j], ib2_ref[j], act)

    X = img_ref[...].astype(jnp.bfloat16)
    P = [X[i].reshape(3, 14, 16, 14, 16).transpose(0, 1, 3, 2, 4)
         .reshape(3, _SEQ, 256) for i in range(_IMB)]
    P = jnp.concatenate(P, axis=1)
    ea = jnp.dot(P[0], tnwT_ref[0], preferred_element_type=jnp.float32)
    ha = jnp.dot(P[0], tW1_ref[0], preferred_element_type=jnp.float32)
    for c in (1, 2):
        ea = ea + jnp.dot(P[c], tnwT_ref[c], preferred_element_type=jnp.float32)
        ha = ha + jnp.dot(P[c], tW1_ref[c], preferred_element_type=jnp.float32)
    e = jax.nn.sigmoid(ea + tnb_ref[...])
    t0 = jnp.dot(e, tree[0:8]) + tree[24:25]
    t1 = jnp.dot(e, tree[8:16]) + tree[25:26]
    t2 = jnp.dot(e, tree[16:24]) + tree[26:27]
    mix = t0 * t1 * t2
    h = jax.nn.relu(ha + tb1_ref[...])
    hm = _mixmul(h, mix)
    x = jnp.dot(hm, tW2_ref[...], preferred_element_type=jnp.float32)
    x = x + jnp.dot(mix, tb2_ref[...])
    x = x + pe_ref[...]
    for bb, base in enumerate((0, 4)):
        xn = _layernorm(x)
        e3 = jax.nn.sigmoid(_dot16(xn, qnwT_ref[bb]) + qnb_ref[bb])
        t0 = jnp.dot(e3, T3[0:24]) + T3[72:73]
        t1 = jnp.dot(e3, T3[24:48]) + T3[73:74]
        t2 = jnp.dot(e3, T3[48:72]) + T3[74:75]
        mix3 = t0 * t1 * t2
        H = jax.nn.relu(_dot16(xn, qW1_ref[bb]) + qb1_ref[bb])
        HM = _mixmul(H, mix3)
        z = jnp.dot(mix3, qb2d_ref[bb])
        q = (jnp.dot(HM[:, 0 * _HID:1 * _HID], iW2_ref[base + 0],
                     preferred_element_type=jnp.float32)
             + z[:, 0 * _LATENT:1 * _LATENT]).astype(jnp.bfloat16)
        k = (jnp.dot(HM[:, 1 * _HID:2 * _HID], iW2_ref[base + 1],
                     preferred_element_type=jnp.float32)
             + z[:, 1 * _LATENT:2 * _LATENT]).astype(jnp.bfloat16)
        v = (jnp.dot(HM[:, 2 * _HID:3 * _HID], iW2_ref[base + 2],
                     preferred_element_type=jnp.float32)
             + z[:, 2 * _LATENT:3 * _LATENT]).astype(jnp.bfloat16)
        avs = []
        for i in range(_IMB):
            sl = slice(i * _SEQ, (i + 1) * _SEQ)
            s = jax.lax.dot_general(q[sl], k[sl], (((1,), (1,)), ((), ())),
                                    preferred_element_type=jnp.float32)
            s = s / (_LATENT ** 0.5)
            m = jnp.max(s, axis=-1, keepdims=True)
            p = jnp.exp(s - m)
            att = p / jnp.sum(p, axis=-1, keepdims=True)
            avs.append(_dot16(att, v[sl]))
        x1 = xn + jnp.concatenate(avs, axis=0)
        x = x1 + fff_j(x1, base + 3, _gelu)
    out_ref[...] = jnp.mean(x.reshape(_IMB, _SEQ, _LATENT), axis=1,
                            keepdims=True)


def _head_body(x_ref, tree_ref, E_ref, nwT_ref, nb_ref, W1_ref, b1_ref,
               W2_ref, b2_ref, out_ref):
    out_ref[...] = _fff(x_ref[...], tree_ref[...], E_ref[...], nwT_ref[...],
                        nb_ref[...], W1_ref[...], b1_ref[...], W2_ref[...],
                        b2_ref[...], jax.nn.relu)


def _full(arr):
    nd = len(arr.shape)
    return pl.BlockSpec(arr.shape, lambda i, _nd=nd: (0,) * _nd)


def _prep_fff(nw, nb, w1, b1, w2, b2, in_w):
    nwT = jnp.swapaxes(jnp.pad(nw, [(0, 0)] * (nw.ndim - 2) + [(0, 1), (0, 0)]),
                       -1, -2)
    nbr = jnp.swapaxes(jnp.pad(nb, [(0, 0)] * (nb.ndim - 2) + [(0, 1), (0, 0)]),
                       -1, -2)
    W1 = jnp.swapaxes(w1, -3, -2).reshape(w1.shape[:-3] + (in_w, _HID))
    b1r = b1.reshape(b1.shape[:-2] + (1, _HID))
    W2 = w2.reshape(w2.shape[:-3] + (_HID, w2.shape[-1]))
    return (nwT.astype(jnp.bfloat16), nbr, W1.astype(jnp.bfloat16), b1r,
            W2.astype(jnp.bfloat16), b2)


def kernel(imgs, tok_nw, tok_nb, tok_w1, tok_b1, tok_w2, tok_b2,
           inn_nw, inn_nb, inn_w1, inn_b1, inn_w2, inn_b2,
           out_nw, out_nb, out_w1, out_b1, out_w2, out_b2):
    B, C, H, W = imgs.shape
    in_w = C * 16 * 16

    treeN, EN = _tree_consts()
    T3N, E3N = _tree3_consts(treeN, EN)
    tree, E = jnp.asarray(treeN), jnp.asarray(EN)
    T3, E3 = jnp.asarray(T3N), jnp.asarray(E3N)

    s = jnp.arange(_SEQ, dtype=jnp.float32)
    pe = jnp.where((jnp.arange(_LATENT) % 2 == 0)[None, :],
                   jnp.sin(s)[:, None], jnp.cos(s)[:, None])

    tok = list(_prep_fff(tok_nw, tok_nb, tok_w1, tok_b1, tok_w2, tok_b2, in_w))
    tok[0] = tok[0].reshape(C, in_w // C, _NL)
    tok[2] = tok[2].reshape(C, in_w // C, _HID)
    inn = _prep_fff(inn_nw, inn_nb, inn_w1, inn_b1, inn_w2, inn_b2, _LATENT)
    out = _prep_fff(out_nw, out_nb, out_w1, out_b1, out_w2, out_b2, _LATENT)

    qnwT = jnp.stack([jnp.concatenate([inn[0][j] for j in ids], axis=-1)
                      for ids in ((0, 1, 2), (4, 5, 6))])
    qnb = jnp.stack([jnp.concatenate([inn[1][j] for j in ids], axis=-1)
                     for ids in ((0, 1, 2), (4, 5, 6))])
    qW1 = jnp.stack([jnp.concatenate([inn[2][j] for j in ids], axis=-1)
                     for ids in ((0, 1, 2), (4, 5, 6))])
    qb1 = jnp.stack([jnp.concatenate([inn[3][j] for j in ids], axis=-1)
                     for ids in ((0, 1, 2), (4, 5, 6))])
    z24 = jnp.zeros((_NL, _LATENT), jnp.float32)
    qb2d = jnp.stack(
        [jnp.block([[inn[5][i0], z24, z24],
                    [z24, inn[5][i0 + 1], z24],
                    [z24, z24, inn[5][i0 + 2]]]) for i0 in (0, 4)])

    pe4 = jnp.tile(pe, (_IMB, 1))
    xm = pl.pallas_call(
        _mega_body,
        grid=(B // _IMB,),
        in_specs=[pl.BlockSpec((_IMB, C, H, W), lambda i: (i, 0, 0, 0)),
                  _full(pe4), _full(tree), _full(E), _full(T3), _full(E3)] +
                 [_full(a) for a in tok] + [_full(a) for a in inn] +
                 [_full(a) for a in (qnwT, qnb, qW1, qb1, qb2d)],
        out_specs=pl.BlockSpec((_IMB, 1, _LATENT), lambda i: (i, 0, 0)),
        out_shape=jax.ShapeDtypeStruct((B, 1, _LATENT), jnp.float32),
        compiler_params=pltpu.CompilerParams(
            dimension_semantics=("parallel",)),
    )(imgs, pe4, tree, E, T3, E3, *tok, *inn, qnwT, qnb, qW1, qb1, qb2d)
    xm = xm.reshape(B, _LATENT)

    y = pl.pallas_call(
        _head_body,
        grid=(1,),
        in_specs=[_full(xm), _full(tree), _full(E)] + [_full(a) for a in out],
        out_specs=_full(jax.ShapeDtypeStruct((B, out_w2.shape[-1]), jnp.float32)),
        out_shape=jax.ShapeDtypeStruct((B, out_w2.shape[-1]), jnp.float32),
    )(xm, tree, E, *out)
    return y
